# baseline (speedup 1.0000x reference)
import functools

import jax
import jax.numpy as jnp
from jax.experimental import pallas as pl
from jax.experimental.pallas import tpu as pltpu

_NK = 11
_C = 128
_LOG2E = 1.4426950408889634


def _rbf_mus(n):
    l = [1.0]
    bs = 2.0 / (n - 1)
    l.append(1 - bs / 2)
    for i in range(1, n - 1):
        l.append(l[i] - bs)
    return l


def _rbf_sigmas(n):
    bs = 2.0 / (n - 1)
    return [0.001] + [0.5 * bs] * (n - 1)


_MUS = _rbf_mus(_NK)
_C2S = [-_LOG2E / (2.0 * s * s) for s in _rbf_sigmas(_NK)]


def _shift_up(x, s):
    return jnp.concatenate(
        [x[s:, :], jnp.zeros((s, x.shape[1]), x.dtype)], axis=0)


def _body(qde_ref, qt_ref, dt_ref, w_ref, b_ref, dw_ref, o_ref,
          *, q_len, d_len):
    qde = qde_ref[0]
    qe = qde[:q_len, :]
    de = qde[q_len:, :]
    w = w_ref[...]

    pq = jnp.dot(qe, w, preferred_element_type=jnp.float32)
    pd = jnp.dot(de, w, preferred_element_type=jnp.float32)

    def grams(p):
        b1 = b_ref[:, 0:_C]
        b2 = b_ref[:, _C:2 * _C]
        b3 = b_ref[:, 2 * _C:3 * _C]
        y1 = jnp.maximum(p[:, 0:_C] + b1, 0.0)
        y2 = jnp.maximum(
            p[:, _C:2 * _C] + _shift_up(p[:, 2 * _C:3 * _C], 1) + b2, 0.0)
        y3 = jnp.maximum(
            p[:, 3 * _C:4 * _C] + _shift_up(p[:, 4 * _C:5 * _C], 1)
            + _shift_up(p[:, 5 * _C:6 * _C], 2) + b3, 0.0)
        return (y1, y2, y3)

    def norm(y):
        n = jnp.sqrt(jnp.sum(y * y, axis=1, keepdims=True))
        return y / (n + 1e-13)

    qg = [norm(y) for y in grams(pq)]
    dg = [norm(y) for y in grams(pd)]

    qm = (qt_ref[0] > 0).astype(jnp.float32)
    dm = (dt_ref[0] > 0).astype(jnp.float32)
    qdm = qm * dm

    acc = jnp.zeros((q_len, 1), jnp.float32)
    for p_idx in range(9):
        qi, dj = p_idx // 3, p_idx % 3
        cos = jax.lax.dot_general(
            qg[qi], dg[dj], (((1,), (1,)), ((), ())),
            preferred_element_type=jnp.float32)
        cosm = cos * qdm
        cols = []
        for k in range(_NK):
            t = cosm - _MUS[k]
            e = jnp.exp2(t * t * _C2S[k]) * qdm
            cols.append(jnp.sum(e, axis=1, keepdims=True))
        s = jnp.concatenate(cols, axis=1)
        lg = jnp.log(jnp.maximum(s, 1e-10)) * 0.01
        acc = acc + jnp.sum(lg * dw_ref[p_idx:p_idx + 1, :], axis=1,
                            keepdims=True)
    o_ref[0] = jnp.sum(acc * qm, keepdims=True)


def kernel(query_tokens, document_tokens, embedding,
           conv_w1, conv_w2, conv_w3, conv_b1, conv_b2, conv_b3, dense_w):
    b, q_len = query_tokens.shape
    _, d_len = document_tokens.shape
    e_dim = embedding.shape[1]

    tok_all = jnp.concatenate([query_tokens, document_tokens], axis=1)
    qd_emb = jnp.take(embedding, tok_all, axis=0)

    w_cat = jnp.concatenate([
        conv_w1[:, :, 0].T,
        conv_w2[:, :, 0].T, conv_w2[:, :, 1].T,
        conv_w3[:, :, 0].T, conv_w3[:, :, 1].T, conv_w3[:, :, 2].T,
    ], axis=1)
    b_cat = jnp.concatenate([conv_b1, conv_b2, conv_b3])[None, :]
    dw = dense_w.reshape(9, _NK)

    qt = query_tokens.reshape(b, q_len, 1)
    dt = document_tokens.reshape(b, 1, d_len)

    out = pl.pallas_call(
        functools.partial(_body, q_len=q_len, d_len=d_len),
        grid=(b,),
        in_specs=[
            pl.BlockSpec((1, q_len + d_len, e_dim), lambda i: (i, 0, 0)),
            pl.BlockSpec((1, q_len, 1), lambda i: (i, 0, 0)),
            pl.BlockSpec((1, 1, d_len), lambda i: (i, 0, 0)),
            pl.BlockSpec((e_dim, 6 * _C), lambda i: (0, 0)),
            pl.BlockSpec((1, 3 * _C), lambda i: (0, 0)),
            pl.BlockSpec((9, _NK), lambda i: (0, 0)),
        ],
        out_specs=pl.BlockSpec((1, 1, 1), lambda i: (i, 0, 0)),
        out_shape=jax.ShapeDtypeStruct((b, 1, 1), jnp.float32),
        compiler_params=pltpu.CompilerParams(
            dimension_semantics=("parallel",),
        ),
    )(qd_emb, qt, dt, w_cat, b_cat, dw)
    return out.reshape(b)


# speedup vs baseline: 1.0473x; 1.0473x over previous
import functools

import jax
import jax.numpy as jnp
from jax.experimental import pallas as pl
from jax.experimental.pallas import tpu as pltpu

_NK = 11
_C = 128
_LOG2E = 1.4426950408889634


def _rbf_mus(n):
    l = [1.0]
    bs = 2.0 / (n - 1)
    l.append(1 - bs / 2)
    for i in range(1, n - 1):
        l.append(l[i] - bs)
    return l


def _rbf_sigmas(n):
    bs = 2.0 / (n - 1)
    return [0.001] + [0.5 * bs] * (n - 1)


_MUS = _rbf_mus(_NK)
_C2S = [-_LOG2E / (2.0 * s * s) for s in _rbf_sigmas(_NK)]


def _shift_up(x, s):
    return jnp.concatenate(
        [x[s:, :], jnp.zeros((s, x.shape[1]), x.dtype)], axis=0)


def _body(qde_ref, qt_ref, dt_ref, w_ref, b_ref, dw_ref, o_ref,
          *, q_len, d_len):
    qde = qde_ref[...]
    qe = qde[:q_len, :]
    de = qde[q_len:q_len + d_len, :]
    w = w_ref[...]

    pq = jnp.dot(qe, w, preferred_element_type=jnp.float32)
    pd = jnp.dot(de, w, preferred_element_type=jnp.float32)

    def grams(p):
        b1 = b_ref[:, 0:_C]
        b2 = b_ref[:, _C:2 * _C]
        b3 = b_ref[:, 2 * _C:3 * _C]
        y1 = jnp.maximum(p[:, 0:_C] + b1, 0.0)
        y2 = jnp.maximum(
            p[:, _C:2 * _C] + _shift_up(p[:, 2 * _C:3 * _C], 1) + b2, 0.0)
        y3 = jnp.maximum(
            p[:, 3 * _C:4 * _C] + _shift_up(p[:, 4 * _C:5 * _C], 1)
            + _shift_up(p[:, 5 * _C:6 * _C], 2) + b3, 0.0)
        return (y1, y2, y3)

    def norm(y):
        n = jnp.sqrt(jnp.sum(y * y, axis=1, keepdims=True))
        return y / (n + 1e-13)

    qg = [norm(y) for y in grams(pq)]
    dg = [norm(y) for y in grams(pd)]

    qm = (qt_ref[0] > 0).astype(jnp.float32)
    dm = (dt_ref[0] > 0).astype(jnp.float32)
    qdm = qm * dm

    acc = jnp.zeros((q_len, 1), jnp.float32)
    for p_idx in range(9):
        qi, dj = p_idx // 3, p_idx % 3
        cos = jax.lax.dot_general(
            qg[qi], dg[dj], (((1,), (1,)), ((), ())),
            preferred_element_type=jnp.float32)
        cosm = cos * qdm
        cols = []
        for k in range(_NK):
            t = cosm - _MUS[k]
            e = jnp.exp2(t * t * _C2S[k]) * qdm
            cols.append(jnp.sum(e, axis=1, keepdims=True))
        s = jnp.concatenate(cols, axis=1)
        lg = jnp.log(jnp.maximum(s, 1e-10)) * 0.01
        acc = acc + jnp.sum(lg * dw_ref[p_idx:p_idx + 1, :], axis=1,
                            keepdims=True)
    o_ref[0] = jnp.sum(acc * qm, keepdims=True)


def kernel(query_tokens, document_tokens, embedding,
           conv_w1, conv_w2, conv_w3, conv_b1, conv_b2, conv_b3, dense_w):
    b, q_len = query_tokens.shape
    _, d_len = document_tokens.shape
    e_dim = embedding.shape[1]

    row = q_len + d_len
    row_pad = (-row) % 8
    tok_all = jnp.concatenate(
        [query_tokens, document_tokens,
         jnp.zeros((b, row_pad), query_tokens.dtype)], axis=1).reshape(-1)
    qd_emb = jnp.take(embedding, tok_all, axis=0)

    w_cat = jnp.concatenate([
        conv_w1[:, :, 0].T,
        conv_w2[:, :, 0].T, conv_w2[:, :, 1].T,
        conv_w3[:, :, 0].T, conv_w3[:, :, 1].T, conv_w3[:, :, 2].T,
    ], axis=1)
    b_cat = jnp.concatenate([conv_b1, conv_b2, conv_b3])[None, :]
    dw = dense_w.reshape(9, _NK)

    qt = query_tokens.reshape(b, q_len, 1)
    dt = document_tokens.reshape(b, 1, d_len)

    out = pl.pallas_call(
        functools.partial(_body, q_len=q_len, d_len=d_len),
        grid=(b,),
        in_specs=[
            pl.BlockSpec((row + row_pad, e_dim), lambda i: (i, 0)),
            pl.BlockSpec((1, q_len, 1), lambda i: (i, 0, 0)),
            pl.BlockSpec((1, 1, d_len), lambda i: (i, 0, 0)),
            pl.BlockSpec((e_dim, 6 * _C), lambda i: (0, 0)),
            pl.BlockSpec((1, 3 * _C), lambda i: (0, 0)),
            pl.BlockSpec((9, _NK), lambda i: (0, 0)),
        ],
        out_specs=pl.BlockSpec((1, 1, 1), lambda i: (i, 0, 0)),
        out_shape=jax.ShapeDtypeStruct((b, 1, 1), jnp.float32),
        compiler_params=pltpu.CompilerParams(
            dimension_semantics=("parallel",),
        ),
    )(qd_emb, qt, dt, w_cat, b_cat, dw)
    return out.reshape(b)
